# baseline (speedup 1.0000x reference)
import functools

import jax
import jax.numpy as jnp
from jax import lax
from jax.experimental import pallas as pl
from jax.experimental.pallas import tpu as pltpu
from jax.experimental.pallas import tpu_sc as plsc

N_NODES = 10000
E_EDGES = 160000
E_LABEL = 8192
D_IN = 256
H1 = 128
H2 = 64

NC = 2
NS = 16
NP = 10240
RPT = NP // NS
FW = 128
BLK = 1000
GRID = N_NODES // BLK

_mesh = plsc.VectorSubcoreMesh(
    core_axis_name="c", subcore_axis_name="s", num_cores=NC, num_subcores=NS)
_sc_params = pltpu.CompilerParams(needs_layout_passes=False)


def _tc1_body(x_ref, w_ref, b_ref, a_ref, hx_ref, av_ref):
    h = jnp.dot(x_ref[...], w_ref[...], preferred_element_type=jnp.float32)
    h = h + b_ref[...]
    hx_ref[...] = h
    av_ref[...] = jnp.dot(h, a_ref[...], preferred_element_type=jnp.float32)


def _tc1(x, w1, b1, a1pad):
    return pl.pallas_call(
        _tc1_body,
        grid=(GRID,),
        in_specs=[
            pl.BlockSpec((BLK, D_IN), lambda i: (i, 0)),
            pl.BlockSpec((D_IN, H1), lambda i: (0, 0)),
            pl.BlockSpec((1, H1), lambda i: (0, 0)),
            pl.BlockSpec((H1, 128), lambda i: (0, 0)),
        ],
        out_specs=[
            pl.BlockSpec((BLK, H1), lambda i: (i, 0)),
            pl.BlockSpec((BLK, 128), lambda i: (i, 0)),
        ],
        out_shape=[
            jax.ShapeDtypeStruct((N_NODES, H1), jnp.float32),
            jax.ShapeDtypeStruct((N_NODES, 128), jnp.float32),
        ],
    )(x, w1, b1, a1pad)


def _make_tc2(F, klin_out, Go):
    def body(agg_ref, kw_ref, kb_ref, w2_ref, b2_ref, cs_ref, g_ref):
        i = pl.program_id(0)

        @pl.when(i == 0)
        def _():
            cs_ref[...] = jnp.zeros((2, klin_out), jnp.float32)

        col = lax.broadcasted_iota(jnp.int32, (BLK, FW), 1)
        for t in (0, 1):
            a = jnp.maximum(agg_ref[t], 0.0)
            if F < FW:
                a = jnp.where(col < F, a, 0.0)
            z = jnp.tanh(
                jnp.dot(a, kw_ref[...], preferred_element_type=jnp.float32)
                + kb_ref[...])
            cs_ref[t:t + 1, :] += jnp.sum(z, axis=0, keepdims=True)
            g_ref[t, :, :] = jnp.dot(
                a, w2_ref[...], preferred_element_type=jnp.float32) + b2_ref[...]

    def call(agg, kwe, kb, w2e, b2e):
        return pl.pallas_call(
            body,
            grid=(GRID,),
            in_specs=[
                pl.BlockSpec((2, BLK, FW), lambda i: (0, i, 0)),
                pl.BlockSpec((FW, klin_out), lambda i: (0, 0)),
                pl.BlockSpec((1, klin_out), lambda i: (0, 0)),
                pl.BlockSpec((FW, Go), lambda i: (0, 0)),
                pl.BlockSpec((1, Go), lambda i: (0, 0)),
            ],
            out_specs=[
                pl.BlockSpec((2, klin_out), lambda i: (0, 0)),
                pl.BlockSpec((2, BLK, Go), lambda i: (0, i, 0)),
            ],
            out_shape=[
                jax.ShapeDtypeStruct((2, klin_out), jnp.float32),
                jax.ShapeDtypeStruct((2, N_NODES, Go), jnp.float32),
            ],
        )(agg, kwe, kb, w2e, b2e)

    return call


def _make_tc3(Go, Fz, with_av):
    def body(g_ref, cs_ref, q_ref, a_ref, o1_ref, o2_ref):
        s0 = jnp.sum(cs_ref[0:1, :] * q_ref[...]) / N_NODES
        s1 = jnp.sum(cs_ref[1:2, :] * q_ref[...]) / N_NODES
        m = jnp.maximum(s0, s1)
        e0 = jnp.exp(s0 - m)
        e1 = jnp.exp(s1 - m)
        b0 = e0 / (e0 + e1)
        b1 = e1 / (e0 + e1)
        h = b0 * g_ref[0] + b1 * g_ref[1]
        o1_ref[...] = h
        if with_av:
            o2_ref[...] = jnp.dot(
                h, a_ref[...], preferred_element_type=jnp.float32)
        else:
            o2_ref[...] = h * a_ref[...]

    n_aux = 128 if with_av else Go

    def call(g, cs, q, a):
        return pl.pallas_call(
            body,
            grid=(GRID,),
            in_specs=[
                pl.BlockSpec((2, BLK, Go), lambda i: (0, i, 0)),
                pl.BlockSpec((2, Fz), lambda i: (0, 0)),
                pl.BlockSpec((1, Fz), lambda i: (0, 0)),
                pl.BlockSpec((Go, 128) if with_av else (1, Go),
                             lambda i: (0, 0)),
            ],
            out_specs=[
                pl.BlockSpec((BLK, Go), lambda i: (i, 0)),
                pl.BlockSpec((BLK, n_aux), lambda i: (i, 0)),
            ],
            out_shape=[
                jax.ShapeDtypeStruct((N_NODES, Go), jnp.float32),
                jax.ShapeDtypeStruct((N_NODES, n_aux), jnp.float32),
            ],
        )(g, cs, q, a)

    return call


def _make_sc_pass():
    C = 128
    PER = 82
    NB = 64
    NH = NP // 2
    AR = NH + 128
    ART = AR // NS
    NHT = NH // NS

    @functools.partial(
        pl.kernel,
        mesh=_mesh,
        compiler_params=_sc_params,
        out_type=[jax.ShapeDtypeStruct((NC, NP, FW), jnp.float32),
                  jax.ShapeDtypeStruct((NC * NS * NP,), jnp.float32)],
        scratch_types=[
            pltpu.VMEM((C,), jnp.int32),
            pltpu.VMEM((C,), jnp.int32),
            pltpu.VMEM((C,), jnp.int32),
            pltpu.VMEM((C, FW), jnp.float32),
            pltpu.VMEM((NP,), jnp.float32),
            pltpu.VMEM((NP,), jnp.float32),
            pltpu.VMEM((NP,), jnp.float32),
            pltpu.VMEM((NS * 2 * NHT,), jnp.float32),
            pltpu.VMEM((2 * NHT,), jnp.float32),
            pltpu.VMEM((NB, FW), jnp.float32),
            pltpu.VMEM_SHARED((AR, FW), jnp.float32),
            pltpu.SemaphoreType.DMA,
        ],
    )
    def k(hx, src1, dst1, asv1, adv1, src2, dst2, asv2, adv2, zrows,
          out, out_den,
          src_v, dst_v, dstl_v, rows_v, asv_v, adv_v, den_v, dred_v, dinv_v,
          nbuf_v, acc, sem):
        c = lax.axis_index("c")
        s = lax.axis_index("s")
        pltpu.sync_copy(zrows.at[pl.ds(s * ART, ART)],
                        acc.at[pl.ds(s * ART, ART)])

        def zd(i, carry):
            den_v[pl.ds(i * 16, 16)] = jnp.zeros((16,), jnp.float32)
            return carry

        lax.fori_loop(0, NP // 16, zd, 0)
        lanes = lax.broadcasted_iota(jnp.int32, (16,), 0)

        for cc, (asv, adv) in ((0, (asv1, adv1)), (1, (asv2, adv2))):
            @pl.when(c == cc)
            def _(asv=asv, adv=adv):
                pltpu.sync_copy(asv, asv_v.at[pl.ds(0, N_NODES)])
                pltpu.sync_copy(adv, adv_v.at[pl.ds(0, N_NODES)])

        def zt(i, carry):
            asv_v[pl.ds(N_NODES + i * 16, 16)] = jnp.zeros((16,), jnp.float32)
            adv_v[pl.ds(N_NODES + i * 16, 16)] = jnp.zeros((16,), jnp.float32)
            return carry

        lax.fori_loop(0, (NP - N_NODES) // 16, zt, 0)
        plsc.subcore_barrier()

        def sweep(half):
            for cc, (srcd, dstd) in ((0, (src1, dst1)), (1, (src2, dst2))):
                @pl.when(c == cc)
                def _(srcd=srcd, dstd=dstd):
                    def chunk(kk, carry):
                        ci = s + kk * NS

                        @pl.when(ci >= 0)
                        def _():
                            base = ci * C
                            pltpu.sync_copy(srcd.at[pl.ds(base, C)], src_v)
                            pltpu.sync_copy(dstd.at[pl.ds(base, C)], dst_v)
                            pltpu.async_copy(hx.at[src_v], rows_v, sem).wait()

                            def blk(b, carry2):
                                sidx = src_v[pl.ds(b * 16, 16)]
                                didx = dst_v[pl.ds(b * 16, 16)]
                                al = (plsc.load_gather(asv_v, [sidx])
                                      + plsc.load_gather(adv_v, [didx]))
                                al = jnp.where(al > 0, al, al * 0.2)
                                exv = jnp.exp(al)
                                dstl = didx - half * NH
                                dstl = jnp.where(
                                    jnp.logical_and(dstl >= 0, dstl < NH),
                                    dstl, NH + (didx & 127))
                                dstl_v[pl.ds(b * 16, 16)] = dstl
                                for j in range(16):
                                    if half == 0:
                                        plsc.addupdate_scatter(
                                            den_v, [didx], exv, mask=lanes == j)
                                    scv = jnp.full((16,), exv[j], jnp.float32)
                                    e = b * 16 + j
                                    for fb in range(FW // 16):
                                        rows_v[e, pl.ds(fb * 16, 16)] = (
                                            rows_v[e, pl.ds(fb * 16, 16)] * scv)
                                return carry2

                            lax.fori_loop(0, C // 16, blk, 0)
                            pltpu.sync_copy(rows_v, acc.at[dstl_v], add=True)

                        return carry

                    lax.fori_loop(0, PER, chunk, 0)

        def normalize(half):
            def norm(ib, carry):
                r0 = s * NHT + ib * NB
                pltpu.sync_copy(acc.at[pl.ds(r0, NB)], nbuf_v)
                for jb in range(NB // 16):
                    iv = dinv_v[pl.ds(half * NHT + ib * NB + jb * 16, 16)]
                    for j in range(16):
                        row = jb * 16 + j
                        scv = jnp.full((16,), iv[j], jnp.float32)
                        for fb in range(FW // 16):
                            nbuf_v[row, pl.ds(fb * 16, 16)] = (
                                nbuf_v[row, pl.ds(fb * 16, 16)] * scv)
                pltpu.sync_copy(
                    nbuf_v, out.at[c, pl.ds(half * NH + r0, NB)])
                return carry

            lax.fori_loop(0, NHT // NB, norm, 0)

        sweep(0)
        plsc.subcore_barrier()
        pltpu.sync_copy(den_v, out_den.at[pl.ds((c * NS + s) * NP, NP)])
        plsc.subcore_barrier()
        for t in range(NS):
            pltpu.sync_copy(
                out_den.at[pl.ds((c * NS + t) * NP + s * NHT, NHT)],
                dred_v.at[pl.ds(t * 2 * NHT, NHT)])
            pltpu.sync_copy(
                out_den.at[pl.ds((c * NS + t) * NP + NH + s * NHT, NHT)],
                dred_v.at[pl.ds(t * 2 * NHT + NHT, NHT)])

        def dredsum(i, carry):
            tot = dred_v[pl.ds(i * 16, 16)]
            for t in range(1, NS):
                tot = tot + dred_v[pl.ds(t * 2 * NHT + i * 16, 16)]
            dinv_v[pl.ds(i * 16, 16)] = 1.0 / (tot + 1e-16)
            return carry

        lax.fori_loop(0, 2 * NHT // 16, dredsum, 0)
        normalize(0)
        plsc.subcore_barrier()
        pltpu.sync_copy(zrows.at[pl.ds(s * ART, ART)],
                        acc.at[pl.ds(s * ART, ART)])
        plsc.subcore_barrier()
        sweep(1)
        plsc.subcore_barrier()
        normalize(1)

    return k


def _link_kernel():
    C = 128
    NCH = E_LABEL // C
    PER = NCH // (NC * NS)

    @functools.partial(
        pl.kernel,
        mesh=_mesh,
        compiler_params=_sc_params,
        out_type=jax.ShapeDtypeStruct((2, E_LABEL), jnp.float32),
        scratch_types=[
            pltpu.VMEM((C,), jnp.int32),
            pltpu.VMEM((C,), jnp.int32),
            pltpu.VMEM((C, FW), jnp.float32),
            pltpu.VMEM((C, FW), jnp.float32),
            pltpu.VMEM((C,), jnp.float32),
            pltpu.VMEM((16,), jnp.float32),
            pltpu.SemaphoreType.DMA,
        ],
    )
    def k(w, eli1, eli2, pbv, out,
          hidx, tidx, ur, vr, outv, pb_v, sem):
        c = lax.axis_index("c")
        s = lax.axis_index("s")
        wid = s * NC + c
        pltpu.sync_copy(pbv, pb_v)
        for t, eli in ((0, eli1), (1, eli2)):
            def chunk(kk, carry, eli=eli, t=t):
                ci = wid * PER + kk
                base = ci * C
                pltpu.sync_copy(eli.at[pl.ds(base, C)], hidx)
                pltpu.sync_copy(eli.at[pl.ds(E_LABEL + base, C)], tidx)
                pltpu.async_copy(w.at[hidx], ur, sem).wait()
                pltpu.async_copy(w.at[tidx], vr, sem).wait()

                def blk(b, carry2):
                    eidx = lax.broadcasted_iota(jnp.int32, (16,), 0) + b * 16

                    def ff(f, acc):
                        fv = jnp.full((16,), f, jnp.int32)
                        uu = plsc.load_gather(ur, [eidx, fv])
                        vv = plsc.load_gather(vr, [eidx, fv + H2])
                        return acc + uu * vv

                    acc = lax.fori_loop(0, H2, ff,
                                        jnp.zeros((16,), jnp.float32),
                                        unroll=4)
                    logit = acc + pb_v[...]
                    outv[pl.ds(b * 16, 16)] = 1.0 / (1.0 + jnp.exp(-logit))
                    return carry2

                lax.fori_loop(0, C // 16, blk, 0)
                pltpu.sync_copy(outv, out.at[t, pl.ds(base, C)])
                return carry

            lax.fori_loop(0, PER, chunk, 0)

    return k


_sc_pass = _make_sc_pass()
_tc2_l1 = _make_tc2(H1, H1, FW)
_tc2_l2 = _make_tc2(H2, H2, H2)
_tc3_l1 = _make_tc3(FW, H1, with_av=True)
_tc3_l2 = _make_tc3(H2, H2, with_av=True)
_link = _link_kernel()


def kernel(x, edge_index_e1, edge_index_e2, edge_label_index_e1,
           edge_label_index_e2, snap,
           proj_W1, proj_b1, a_src1_e1, a_dst1_e1, a_src1_e2, a_dst1_e2,
           klin_W1, klin_b1, q1,
           proj_W2, proj_b2, a_src2_e1, a_dst2_e1, a_src2_e2, a_dst2_e2,
           klin_W2, klin_b2, q2, post_W, post_b):
    f32 = jnp.float32
    b1r = proj_b1[None]
    a1pad = (jnp.zeros((H1, 128), f32)
             .at[:, 0].set(a_src1_e1).at[:, 1].set(a_dst1_e1)
             .at[:, 2].set(a_src1_e2).at[:, 3].set(a_dst1_e2))
    kb1 = klin_b1[None]
    w2e = jnp.zeros((FW, FW), f32).at[:H1, :H2].set(proj_W2)
    b2e = jnp.zeros((1, FW), f32).at[0, :H2].set(proj_b2)
    a2pad = (jnp.zeros((FW, 128), f32)
             .at[:H2, 0].set(a_src2_e1).at[:H2, 1].set(a_dst2_e1)
             .at[:H2, 2].set(a_src2_e2).at[:H2, 3].set(a_dst2_e2))
    kw2e = jnp.zeros((FW, H2), f32).at[:H2].set(klin_W2)
    kb2 = klin_b2[None]
    eyep = jnp.zeros((FW, H2), f32).at[:H2].set(jnp.eye(H2, dtype=f32))
    q1r = q1[None]
    q2r = q2[None]
    pw = post_W.sum(-1)
    uvpack = jnp.concatenate([jnp.eye(H2, dtype=f32), jnp.diag(pw)], axis=1)
    pbv = jnp.full((16,), post_b.sum(), f32)
    zrows = jnp.zeros((NP // 2 + 128, FW), f32)

    EPAD = 16 * 82 * 128
    _pad_i = jnp.arange(EPAD - E_EDGES, dtype=jnp.int32)
    _pad_src = (_pad_i * 997) % N_NODES
    _pad_dst = N_NODES + (_pad_i % (10240 - N_NODES))
    def padsrc(a):
        return jnp.concatenate([a, _pad_src])
    def paddst(a):
        return jnp.concatenate([a, _pad_dst])
    s1p, d1p = padsrc(edge_index_e1[0]), paddst(edge_index_e1[1])
    s2p, d2p = padsrc(edge_index_e2[0]), paddst(edge_index_e2[1])
    h1, av1 = _tc1(x, proj_W1, b1r, a1pad)
    agg1, _den1 = _sc_pass(h1, s1p, d1p, av1[:, 0], av1[:, 1],
                           s2p, d2p, av1[:, 2], av1[:, 3], zrows)
    cs1, g1 = _tc2_l1(agg1, klin_W1, kb1, w2e, b2e)
    hx2, av2 = _tc3_l1(g1, cs1, q1r, a2pad)

    agg2, _den2 = _sc_pass(hx2, s1p, d1p, av2[:, 0], av2[:, 1],
                           s2p, d2p, av2[:, 2], av2[:, 3], zrows)
    cs2, g2 = _tc2_l2(agg2, kw2e, kb2, eyep, jnp.zeros((1, H2), f32))
    _h2, w = _tc3_l2(g2, cs2, q2r, uvpack)

    return _link(w, edge_label_index_e1.reshape(-1),
                 edge_label_index_e2.reshape(-1), pbv)


# speedup vs baseline: 1.2958x; 1.2958x over previous
import functools

import jax
import jax.numpy as jnp
from jax import lax
from jax.experimental import pallas as pl
from jax.experimental.pallas import tpu as pltpu
from jax.experimental.pallas import tpu_sc as plsc

N_NODES = 10000
E_EDGES = 160000
E_LABEL = 8192
D_IN = 256
H1 = 128
H2 = 64

NC = 2
NS = 16
NP = 10240
RPT = NP // NS
FW = 128
BLK = 1000
GRID = N_NODES // BLK

_mesh = plsc.VectorSubcoreMesh(
    core_axis_name="c", subcore_axis_name="s", num_cores=NC, num_subcores=NS)
_sc_params = pltpu.CompilerParams(needs_layout_passes=False)


def _tc1_body(x_ref, w_ref, b_ref, a_ref, hx_ref, av_ref):
    h = jnp.dot(x_ref[...], w_ref[...], preferred_element_type=jnp.float32)
    h = h + b_ref[...]
    hx_ref[...] = h
    av_ref[...] = jnp.dot(h, a_ref[...], preferred_element_type=jnp.float32)


def _tc1(x, w1, b1, a1pad):
    return pl.pallas_call(
        _tc1_body,
        grid=(GRID,),
        in_specs=[
            pl.BlockSpec((BLK, D_IN), lambda i: (i, 0)),
            pl.BlockSpec((D_IN, H1), lambda i: (0, 0)),
            pl.BlockSpec((1, H1), lambda i: (0, 0)),
            pl.BlockSpec((H1, 128), lambda i: (0, 0)),
        ],
        out_specs=[
            pl.BlockSpec((BLK, H1), lambda i: (i, 0)),
            pl.BlockSpec((BLK, 128), lambda i: (i, 0)),
        ],
        out_shape=[
            jax.ShapeDtypeStruct((N_NODES, H1), jnp.float32),
            jax.ShapeDtypeStruct((N_NODES, 128), jnp.float32),
        ],
    )(x, w1, b1, a1pad)


def _make_tc2(F, klin_out, Go):
    def body(agg_ref, kw_ref, kb_ref, w2_ref, b2_ref, cs_ref, g_ref):
        i = pl.program_id(0)

        @pl.when(i == 0)
        def _():
            cs_ref[...] = jnp.zeros((2, klin_out), jnp.float32)

        col = lax.broadcasted_iota(jnp.int32, (BLK, FW), 1)
        for t in (0, 1):
            a = jnp.maximum(agg_ref[t], 0.0)
            if F < FW:
                a = jnp.where(col < F, a, 0.0)
            z = jnp.tanh(
                jnp.dot(a, kw_ref[...], preferred_element_type=jnp.float32)
                + kb_ref[...])
            cs_ref[t:t + 1, :] += jnp.sum(z, axis=0, keepdims=True)
            g_ref[t, :, :] = jnp.dot(
                a, w2_ref[...], preferred_element_type=jnp.float32) + b2_ref[...]

    def call(agg, kwe, kb, w2e, b2e):
        return pl.pallas_call(
            body,
            grid=(GRID,),
            in_specs=[
                pl.BlockSpec((2, BLK, FW), lambda i: (0, i, 0)),
                pl.BlockSpec((FW, klin_out), lambda i: (0, 0)),
                pl.BlockSpec((1, klin_out), lambda i: (0, 0)),
                pl.BlockSpec((FW, Go), lambda i: (0, 0)),
                pl.BlockSpec((1, Go), lambda i: (0, 0)),
            ],
            out_specs=[
                pl.BlockSpec((2, klin_out), lambda i: (0, 0)),
                pl.BlockSpec((2, BLK, Go), lambda i: (0, i, 0)),
            ],
            out_shape=[
                jax.ShapeDtypeStruct((2, klin_out), jnp.float32),
                jax.ShapeDtypeStruct((2, N_NODES, Go), jnp.float32),
            ],
        )(agg, kwe, kb, w2e, b2e)

    return call


def _make_tc3(Go, Fz, with_av):
    def body(g_ref, cs_ref, q_ref, a_ref, o1_ref, o2_ref):
        s0 = jnp.sum(cs_ref[0:1, :] * q_ref[...]) / N_NODES
        s1 = jnp.sum(cs_ref[1:2, :] * q_ref[...]) / N_NODES
        m = jnp.maximum(s0, s1)
        e0 = jnp.exp(s0 - m)
        e1 = jnp.exp(s1 - m)
        b0 = e0 / (e0 + e1)
        b1 = e1 / (e0 + e1)
        h = b0 * g_ref[0] + b1 * g_ref[1]
        o1_ref[...] = h
        if with_av:
            o2_ref[...] = jnp.dot(
                h, a_ref[...], preferred_element_type=jnp.float32)
        else:
            o2_ref[...] = h * a_ref[...]

    n_aux = 128 if with_av else Go

    def call(g, cs, q, a):
        return pl.pallas_call(
            body,
            grid=(GRID,),
            in_specs=[
                pl.BlockSpec((2, BLK, Go), lambda i: (0, i, 0)),
                pl.BlockSpec((2, Fz), lambda i: (0, 0)),
                pl.BlockSpec((1, Fz), lambda i: (0, 0)),
                pl.BlockSpec((Go, 128) if with_av else (1, Go),
                             lambda i: (0, 0)),
            ],
            out_specs=[
                pl.BlockSpec((BLK, Go), lambda i: (i, 0)),
                pl.BlockSpec((BLK, n_aux), lambda i: (i, 0)),
            ],
            out_shape=[
                jax.ShapeDtypeStruct((N_NODES, Go), jnp.float32),
                jax.ShapeDtypeStruct((N_NODES, n_aux), jnp.float32),
            ],
        )(g, cs, q, a)

    return call


def _make_sc_pass():
    C = 128
    PER = 82
    NB = 64
    NH = NP // 2
    AR = NH + 128
    ART = AR // NS
    NHT = NH // NS

    @functools.partial(
        pl.kernel,
        mesh=_mesh,
        compiler_params=_sc_params,
        out_type=[jax.ShapeDtypeStruct((NC, NP, FW), jnp.float32),
                  jax.ShapeDtypeStruct((NC * NS * NP,), jnp.float32)],
        scratch_types=[
            pltpu.VMEM((C,), jnp.int32),
            pltpu.VMEM((C,), jnp.int32),
            pltpu.VMEM((C,), jnp.int32),
            pltpu.VMEM((C,), jnp.int32),
            pltpu.VMEM((C,), jnp.int32),
            pltpu.VMEM((C, FW), jnp.float32),
            pltpu.VMEM((C, FW), jnp.float32),
            pltpu.VMEM((NP,), jnp.float32),
            pltpu.VMEM((NP,), jnp.float32),
            pltpu.VMEM((NP,), jnp.float32),
            pltpu.VMEM((NS * 2 * NHT,), jnp.float32),
            pltpu.VMEM((2 * NHT,), jnp.float32),
            pltpu.VMEM((NB, FW), jnp.float32),
            pltpu.VMEM_SHARED((AR, FW), jnp.float32),
            pltpu.SemaphoreType.DMA,
            pltpu.SemaphoreType.DMA,
        ],
    )
    def k(hx, src1, dst1, asv1, adv1, src2, dst2, asv2, adv2, zrows,
          out, out_den,
          src_a, src_b, dst_a, dst_b, dstl_v, rows_a, rows_b,
          asv_v, adv_v, den_v, dred_v, dinv_v,
          nbuf_v, acc, sem, sem_b):
        c = lax.axis_index("c")
        s = lax.axis_index("s")
        pltpu.sync_copy(zrows.at[pl.ds(s * ART, ART)],
                        acc.at[pl.ds(s * ART, ART)])

        def zd(i, carry):
            den_v[pl.ds(i * 16, 16)] = jnp.zeros((16,), jnp.float32)
            return carry

        lax.fori_loop(0, NP // 16, zd, 0)
        lanes = lax.broadcasted_iota(jnp.int32, (16,), 0)

        for cc, (asv, adv) in ((0, (asv1, adv1)), (1, (asv2, adv2))):
            @pl.when(c == cc)
            def _(asv=asv, adv=adv):
                pltpu.sync_copy(asv, asv_v.at[pl.ds(0, N_NODES)])
                pltpu.sync_copy(adv, adv_v.at[pl.ds(0, N_NODES)])

        def zt(i, carry):
            asv_v[pl.ds(N_NODES + i * 16, 16)] = jnp.zeros((16,), jnp.float32)
            adv_v[pl.ds(N_NODES + i * 16, 16)] = jnp.zeros((16,), jnp.float32)
            return carry

        lax.fori_loop(0, (NP - N_NODES) // 16, zt, 0)
        plsc.subcore_barrier()

        def sweep(half):
            for cc, (srcd, dstd) in ((0, (src1, dst1)), (1, (src2, dst2))):
                @pl.when(c == cc)
                def _(srcd=srcd, dstd=dstd):
                    def fetch(kk, src_v, dst_v, rows_v, gsem):
                        base = (s + kk * NS) * C
                        pltpu.sync_copy(srcd.at[pl.ds(base, C)], src_v)
                        pltpu.sync_copy(dstd.at[pl.ds(base, C)], dst_v)
                        return pltpu.async_copy(hx.at[src_v], rows_v, gsem)

                    def compute(src_v, dst_v, rows_v):
                        if True:
                            def blk(b, carry2):
                                sidx = src_v[pl.ds(b * 16, 16)]
                                didx = dst_v[pl.ds(b * 16, 16)]
                                al = (plsc.load_gather(asv_v, [sidx])
                                      + plsc.load_gather(adv_v, [didx]))
                                al = jnp.where(al > 0, al, al * 0.2)
                                exv = jnp.exp(al)
                                dstl = didx - half * NH
                                dstl = jnp.where(
                                    jnp.logical_and(dstl >= 0, dstl < NH),
                                    dstl, NH + (didx & 127))
                                dstl_v[pl.ds(b * 16, 16)] = dstl
                                for j in range(16):
                                    if half == 0:
                                        plsc.addupdate_scatter(
                                            den_v, [didx], exv, mask=lanes == j)
                                    scv = jnp.full((16,), exv[j], jnp.float32)
                                    e = b * 16 + j
                                    for fb in range(FW // 16):
                                        rows_v[e, pl.ds(fb * 16, 16)] = (
                                            rows_v[e, pl.ds(fb * 16, 16)] * scv)
                                return carry2

                            lax.fori_loop(0, C // 16, blk, 0)
                            pltpu.sync_copy(rows_v, acc.at[dstl_v], add=True)

                    def pair(k2, carry):
                        ha = fetch(2 * k2, src_a, dst_a, rows_a, sem)
                        hb = fetch(2 * k2 + 1, src_b, dst_b, rows_b, sem_b)
                        ha.wait()
                        compute(src_a, dst_a, rows_a)
                        hb.wait()
                        compute(src_b, dst_b, rows_b)
                        return carry

                    lax.fori_loop(0, PER // 2, pair, 0)

        def normalize(half):
            def norm(ib, carry):
                r0 = s * NHT + ib * NB
                pltpu.sync_copy(acc.at[pl.ds(r0, NB)], nbuf_v)
                for jb in range(NB // 16):
                    iv = dinv_v[pl.ds(half * NHT + ib * NB + jb * 16, 16)]
                    for j in range(16):
                        row = jb * 16 + j
                        scv = jnp.full((16,), iv[j], jnp.float32)
                        for fb in range(FW // 16):
                            nbuf_v[row, pl.ds(fb * 16, 16)] = (
                                nbuf_v[row, pl.ds(fb * 16, 16)] * scv)
                pltpu.sync_copy(
                    nbuf_v, out.at[c, pl.ds(half * NH + r0, NB)])
                return carry

            lax.fori_loop(0, NHT // NB, norm, 0)

        sweep(0)
        plsc.subcore_barrier()
        pltpu.sync_copy(den_v, out_den.at[pl.ds((c * NS + s) * NP, NP)])
        plsc.subcore_barrier()
        for t in range(NS):
            pltpu.sync_copy(
                out_den.at[pl.ds((c * NS + t) * NP + s * NHT, NHT)],
                dred_v.at[pl.ds(t * 2 * NHT, NHT)])
            pltpu.sync_copy(
                out_den.at[pl.ds((c * NS + t) * NP + NH + s * NHT, NHT)],
                dred_v.at[pl.ds(t * 2 * NHT + NHT, NHT)])

        def dredsum(i, carry):
            tot = dred_v[pl.ds(i * 16, 16)]
            for t in range(1, NS):
                tot = tot + dred_v[pl.ds(t * 2 * NHT + i * 16, 16)]
            dinv_v[pl.ds(i * 16, 16)] = 1.0 / (tot + 1e-16)
            return carry

        lax.fori_loop(0, 2 * NHT // 16, dredsum, 0)
        normalize(0)
        plsc.subcore_barrier()
        pltpu.sync_copy(zrows.at[pl.ds(s * ART, ART)],
                        acc.at[pl.ds(s * ART, ART)])
        plsc.subcore_barrier()
        sweep(1)
        plsc.subcore_barrier()
        normalize(1)

    return k


def _link_kernel():
    C = 128
    NCH = E_LABEL // C
    PER = NCH // (NC * NS)

    @functools.partial(
        pl.kernel,
        mesh=_mesh,
        compiler_params=_sc_params,
        out_type=jax.ShapeDtypeStruct((2, E_LABEL), jnp.float32),
        scratch_types=[
            pltpu.VMEM((C,), jnp.int32),
            pltpu.VMEM((C,), jnp.int32),
            pltpu.VMEM((C, FW), jnp.float32),
            pltpu.VMEM((C, FW), jnp.float32),
            pltpu.VMEM((C,), jnp.float32),
            pltpu.VMEM((16,), jnp.float32),
            pltpu.SemaphoreType.DMA,
        ],
    )
    def k(w, eli1, eli2, pbv, out,
          hidx, tidx, ur, vr, outv, pb_v, sem):
        c = lax.axis_index("c")
        s = lax.axis_index("s")
        wid = s * NC + c
        pltpu.sync_copy(pbv, pb_v)
        for t, eli in ((0, eli1), (1, eli2)):
            def chunk(kk, carry, eli=eli, t=t):
                ci = wid * PER + kk
                base = ci * C
                pltpu.sync_copy(eli.at[pl.ds(base, C)], hidx)
                pltpu.sync_copy(eli.at[pl.ds(E_LABEL + base, C)], tidx)
                pltpu.async_copy(w.at[hidx], ur, sem).wait()
                pltpu.async_copy(w.at[tidx], vr, sem).wait()

                def blk(b, carry2):
                    eidx = lax.broadcasted_iota(jnp.int32, (16,), 0) + b * 16

                    def ff(f, acc):
                        fv = jnp.full((16,), f, jnp.int32)
                        uu = plsc.load_gather(ur, [eidx, fv])
                        vv = plsc.load_gather(vr, [eidx, fv + H2])
                        return acc + uu * vv

                    acc = lax.fori_loop(0, H2, ff,
                                        jnp.zeros((16,), jnp.float32),
                                        unroll=4)
                    logit = acc + pb_v[...]
                    outv[pl.ds(b * 16, 16)] = 1.0 / (1.0 + jnp.exp(-logit))
                    return carry2

                lax.fori_loop(0, C // 16, blk, 0)
                pltpu.sync_copy(outv, out.at[t, pl.ds(base, C)])
                return carry

            lax.fori_loop(0, PER, chunk, 0)

    return k


_sc_pass = _make_sc_pass()
_tc2_l1 = _make_tc2(H1, H1, FW)
_tc2_l2 = _make_tc2(H2, H2, H2)
_tc3_l1 = _make_tc3(FW, H1, with_av=True)
_tc3_l2 = _make_tc3(H2, H2, with_av=True)
_link = _link_kernel()


def kernel(x, edge_index_e1, edge_index_e2, edge_label_index_e1,
           edge_label_index_e2, snap,
           proj_W1, proj_b1, a_src1_e1, a_dst1_e1, a_src1_e2, a_dst1_e2,
           klin_W1, klin_b1, q1,
           proj_W2, proj_b2, a_src2_e1, a_dst2_e1, a_src2_e2, a_dst2_e2,
           klin_W2, klin_b2, q2, post_W, post_b):
    f32 = jnp.float32
    b1r = proj_b1[None]
    a1pad = (jnp.zeros((H1, 128), f32)
             .at[:, 0].set(a_src1_e1).at[:, 1].set(a_dst1_e1)
             .at[:, 2].set(a_src1_e2).at[:, 3].set(a_dst1_e2))
    kb1 = klin_b1[None]
    w2e = jnp.zeros((FW, FW), f32).at[:H1, :H2].set(proj_W2)
    b2e = jnp.zeros((1, FW), f32).at[0, :H2].set(proj_b2)
    a2pad = (jnp.zeros((FW, 128), f32)
             .at[:H2, 0].set(a_src2_e1).at[:H2, 1].set(a_dst2_e1)
             .at[:H2, 2].set(a_src2_e2).at[:H2, 3].set(a_dst2_e2))
    kw2e = jnp.zeros((FW, H2), f32).at[:H2].set(klin_W2)
    kb2 = klin_b2[None]
    eyep = jnp.zeros((FW, H2), f32).at[:H2].set(jnp.eye(H2, dtype=f32))
    q1r = q1[None]
    q2r = q2[None]
    pw = post_W.sum(-1)
    uvpack = jnp.concatenate([jnp.eye(H2, dtype=f32), jnp.diag(pw)], axis=1)
    pbv = jnp.full((16,), post_b.sum(), f32)
    zrows = jnp.zeros((NP // 2 + 128, FW), f32)

    EPAD = 16 * 82 * 128
    _pad_i = jnp.arange(EPAD - E_EDGES, dtype=jnp.int32)
    _pad_src = (_pad_i * 997) % N_NODES
    _pad_dst = N_NODES + (_pad_i % (10240 - N_NODES))
    def padsrc(a):
        return jnp.concatenate([a, _pad_src])
    def paddst(a):
        return jnp.concatenate([a, _pad_dst])
    s1p, d1p = padsrc(edge_index_e1[0]), paddst(edge_index_e1[1])
    s2p, d2p = padsrc(edge_index_e2[0]), paddst(edge_index_e2[1])
    h1, av1 = _tc1(x, proj_W1, b1r, a1pad)
    agg1, _den1 = _sc_pass(h1, s1p, d1p, av1[:, 0], av1[:, 1],
                           s2p, d2p, av1[:, 2], av1[:, 3], zrows)
    cs1, g1 = _tc2_l1(agg1, klin_W1, kb1, w2e, b2e)
    hx2, av2 = _tc3_l1(g1, cs1, q1r, a2pad)

    agg2, _den2 = _sc_pass(hx2, s1p, d1p, av2[:, 0], av2[:, 1],
                           s2p, d2p, av2[:, 2], av2[:, 3], zrows)
    cs2, g2 = _tc2_l2(agg2, kw2e, kb2, eyep, jnp.zeros((1, H2), f32))
    _h2, w = _tc3_l2(g2, cs2, q2r, uvpack)

    return _link(w, edge_label_index_e1.reshape(-1),
                 edge_label_index_e2.reshape(-1), pbv)
